# Initial kernel scaffold; baseline (speedup 1.0000x reference)
import jax
import jax.numpy as jnp
from jax.experimental import pallas as pl


def kernel(snip_features, topic_embedding, W1, b1, gamma1, beta1, Wt, bt, gammat, betat, Wg1, bg1, Wg2, bg2):
    raise NotImplementedError("write your pallas kernel here")


# speedup vs baseline: 23.6190x; 23.6190x over previous
import jax
import jax.numpy as jnp
from jax.experimental import pallas as pl
from jax.scipy.linalg import block_diag

B, C, T = 4, 256, 2048
TT, TD = 128, 16
K = 6
GCN_G, CONV_G = 32, 4
TB = 512
NT = T // TB
EPS = 1e-5


def _conv_kernel(x_ref, w0_ref, w1_ref, w2_ref, b1_ref, y_ref, st_ref):
    x = x_ref[0]
    y0 = jnp.dot(w0_ref[...], x, preferred_element_type=jnp.float32)
    y1 = jnp.dot(w1_ref[...], x, preferred_element_type=jnp.float32)
    y2 = jnp.dot(w2_ref[...], x, preferred_element_type=jnp.float32)
    z = jnp.zeros((C, 1), jnp.float32)
    y = (y1 + jnp.concatenate([z, y0[:, :-1]], axis=1)
         + jnp.concatenate([y2[:, 1:], z], axis=1) + b1_ref[:, 0:1])
    y_ref[0] = y
    s1 = jnp.sum(y, axis=1, keepdims=True)
    s2 = jnp.sum(y * y, axis=1, keepdims=True)
    st_ref[0] = jnp.pad(jnp.concatenate([s1, s2], axis=1), ((0, 0), (0, 126)))


def _topic_kernel(te_ref, wtd_ref, gbt_ref, mb1_ref, mb2_ref,
                  tf_ref, tm1_ref, tm2_ref, t2_ref):
    wtd = wtd_ref[...]
    bt = gbt_ref[0:1, :]
    gam = gbt_ref[1:2, :]
    bet = gbt_ref[2:3, :]
    ys = []
    s = jnp.zeros((1, C), jnp.float32)
    for b in range(B):
        y = jnp.dot(te_ref[b], wtd, preferred_element_type=jnp.float32) + bt
        ys.append(y)
        s = s + jnp.sum(y, axis=0, keepdims=True)
    mean = s / float(B * TT)
    v = jnp.zeros((1, C), jnp.float32)
    for b in range(B):
        dlt = ys[b] - mean
        v = v + jnp.sum(dlt * dlt, axis=0, keepdims=True)
    var = v / float(B * TT)
    a = gam * jax.lax.rsqrt(var + EPS)
    d = bet - mean * a
    mb1 = mb1_ref[...]
    mb2 = mb2_ref[...]
    for b in range(B):
        f = jnp.maximum(ys[b] * a + d, 0.0)
        tf_ref[b] = f
        tm1_ref[b] = jnp.dot(f, mb1, preferred_element_type=jnp.float32).T
        tm2_ref[b] = jnp.dot(f, mb2, preferred_element_type=jnp.float32).T
        t2 = jnp.sum(f * f, axis=1, keepdims=True)
        t2_ref[b] = jnp.broadcast_to(t2, (TT, TT))


def _gcn_kernel(y_ref, x_ref, tf_ref, tm1_ref, tm2_ref, t2_ref,
                a2_ref, d2_ref, m1_ref, m2_ref, bg1_ref, bg2_ref, out_ref):
    base = jnp.maximum(y_ref[0] * a2_ref[:, 0:1] + d2_ref[:, 0:1], 0.0)
    tf = tf_ref[0]
    t2 = t2_ref[0][:, 0:1]
    iota_s = jax.lax.broadcasted_iota(jnp.int32, (TT, TB), 0)
    for tm_ref, m_ref, bg_ref in ((tm1_ref, m1_ref, bg1_ref),
                                  (tm2_ref, m2_ref, bg2_ref)):
        tm = tm_ref[0]
        x2 = jnp.sum(base * base, axis=0, keepdims=True)
        cross = jnp.dot(tf, base, preferred_element_type=jnp.float32)
        dist = x2 - 2.0 * cross + t2
        run = jnp.full((TT, TB), -jnp.inf, jnp.float32)
        for _ in range(K):
            m = jnp.min(dist, axis=0, keepdims=True)
            j = jnp.min(jnp.where(dist == m, iota_s, TT),
                        axis=0, keepdims=True)
            sel = iota_s == j
            picked = jnp.dot(tm, sel.astype(jnp.float32),
                             preferred_element_type=jnp.float32)
            run = jnp.maximum(run, picked)
            dist = jnp.where(sel, 3.0e38, dist)
        xab = jnp.dot(m_ref[...], base, preferred_element_type=jnp.float32)
        agg = jnp.concatenate([xab[0:128], run - xab[128:256]], axis=0)
        base = jnp.maximum(base + agg + bg_ref[:, 0:1], 0.0)
    out_ref[0] = base + x_ref[0]


def kernel(snip_features, topic_embedding, W1, b1, gamma1, beta1,
           Wt, bt, gammat, betat, Wg1, bg1, Wg2, bg2):
    f32 = jnp.float32

    w0, w1t, w2 = (block_diag(*[W1[64 * g:64 * (g + 1), :, dt] for g in range(CONV_G)])
                   for dt in range(3))
    wtd = block_diag(*[Wt[64 * g:64 * (g + 1), :, 0].T for g in range(CONV_G)])
    maT1 = block_diag(*[Wg1[i] for i in range(16)])
    mbT1 = block_diag(*[Wg1[16 + i] for i in range(16)])
    maT2 = block_diag(*[Wg2[i] for i in range(16)])
    mbT2 = block_diag(*[Wg2[16 + i] for i in range(16)])
    m1 = jnp.concatenate([maT1, mbT1], axis=0)
    m2 = jnp.concatenate([maT2, mbT2], axis=0)
    b1c = jnp.broadcast_to(b1[:, None], (C, 128))
    gbt = jnp.pad(jnp.stack([bt, gammat, betat]), ((0, 5), (0, 0)))
    bg1c = jnp.broadcast_to(bg1[:, None], (C, 128))
    bg2c = jnp.broadcast_to(bg2[:, None], (C, 128))

    y, ystats = pl.pallas_call(
        _conv_kernel,
        grid=(B,),
        in_specs=[
            pl.BlockSpec((1, C, T), lambda b: (b, 0, 0)),
            pl.BlockSpec((C, C), lambda b: (0, 0)),
            pl.BlockSpec((C, C), lambda b: (0, 0)),
            pl.BlockSpec((C, C), lambda b: (0, 0)),
            pl.BlockSpec((C, 128), lambda b: (0, 0)),
        ],
        out_specs=[
            pl.BlockSpec((1, C, T), lambda b: (b, 0, 0)),
            pl.BlockSpec((1, C, 128), lambda b: (b, 0, 0)),
        ],
        out_shape=[
            jax.ShapeDtypeStruct((B, C, T), f32),
            jax.ShapeDtypeStruct((B, C, 128), f32),
        ],
    )(snip_features, w0, w1t, w2, b1c)

    n = float(B * T)
    mean = jnp.sum(ystats[:, :, 0], axis=0) / n
    var = jnp.sum(ystats[:, :, 1], axis=0) / n - mean * mean
    a = gamma1 * jax.lax.rsqrt(var + EPS)
    d = beta1 - mean * a
    a2 = jnp.broadcast_to(a[:, None], (C, 128))
    d2 = jnp.broadcast_to(d[:, None], (C, 128))

    tf, tm1, tm2, t2b = pl.pallas_call(
        _topic_kernel,
        grid=(1,),
        in_specs=[
            pl.BlockSpec((B, TT, TD), lambda i: (0, 0, 0)),
            pl.BlockSpec((TD, C), lambda i: (0, 0)),
            pl.BlockSpec((8, C), lambda i: (0, 0)),
            pl.BlockSpec((C, 128), lambda i: (0, 0)),
            pl.BlockSpec((C, 128), lambda i: (0, 0)),
        ],
        out_specs=[
            pl.BlockSpec((B, TT, C), lambda i: (0, 0, 0)),
            pl.BlockSpec((B, TT, TT), lambda i: (0, 0, 0)),
            pl.BlockSpec((B, TT, TT), lambda i: (0, 0, 0)),
            pl.BlockSpec((B, TT, TT), lambda i: (0, 0, 0)),
        ],
        out_shape=[
            jax.ShapeDtypeStruct((B, TT, C), f32),
            jax.ShapeDtypeStruct((B, TT, TT), f32),
            jax.ShapeDtypeStruct((B, TT, TT), f32),
            jax.ShapeDtypeStruct((B, TT, TT), f32),
        ],
    )(topic_embedding, wtd, gbt, mbT1.T, mbT2.T)

    out = pl.pallas_call(
        _gcn_kernel,
        grid=(B, NT),
        in_specs=[
            pl.BlockSpec((1, C, TB), lambda b, t: (b, 0, t)),
            pl.BlockSpec((1, C, TB), lambda b, t: (b, 0, t)),
            pl.BlockSpec((1, TT, C), lambda b, t: (b, 0, 0)),
            pl.BlockSpec((1, TT, TT), lambda b, t: (b, 0, 0)),
            pl.BlockSpec((1, TT, TT), lambda b, t: (b, 0, 0)),
            pl.BlockSpec((1, TT, TT), lambda b, t: (b, 0, 0)),
            pl.BlockSpec((C, 128), lambda b, t: (0, 0)),
            pl.BlockSpec((C, 128), lambda b, t: (0, 0)),
            pl.BlockSpec((C, C), lambda b, t: (0, 0)),
            pl.BlockSpec((C, C), lambda b, t: (0, 0)),
            pl.BlockSpec((C, 128), lambda b, t: (0, 0)),
            pl.BlockSpec((C, 128), lambda b, t: (0, 0)),
        ],
        out_specs=pl.BlockSpec((1, C, TB), lambda b, t: (b, 0, t)),
        out_shape=jax.ShapeDtypeStruct((B, C, T), f32),
    )(y, snip_features, tf, tm1, tm2, t2b, a2, d2, m1, m2, bg1c, bg2c)

    return out
